# Initial kernel scaffold; baseline (speedup 1.0000x reference)
import jax
import jax.numpy as jnp
from jax.experimental import pallas as pl


def kernel(x, W_dq, W_uq, q_ln_g, q_ln_b, W_dkv, W_ukv, kv_ln_g, kv_ln_b, W_o):
    raise NotImplementedError("write your pallas kernel here")


# speedup vs baseline: 1.0312x; 1.0312x over previous
import jax
import jax.numpy as jnp
from jax.experimental import pallas as pl
from jax.experimental.pallas import tpu as pltpu

D = 2048
S = 2048
H = 16
DH = 128
NOPE = 64
RP = 64
QPD = 1024
KVPD = 1365
CKV_W = KVPD + RP
KV_OUT = D + H * NOPE
BQ = 256
EPS = 1e-5
SCALE = 1.0 / (DH ** 0.5)
F32 = jnp.float32


def _rot_half(x):
    half = x.shape[-1] // 2
    return jnp.concatenate([-x[:, half:], x[:, :half]], axis=-1)


def _q_proj_kernel(x_ref, wdq_ref, wuq_ref, g_ref, b_ref, q_ref):
    cq = jnp.dot(x_ref[...], wdq_ref[...], preferred_element_type=F32)
    m = jnp.mean(cq, axis=-1, keepdims=True)
    d = cq - m
    v = jnp.mean(d * d, axis=-1, keepdims=True)
    cqn = d * jax.lax.rsqrt(v + EPS) * g_ref[...] + b_ref[...]
    q_ref[...] = jnp.dot(cqn, wuq_ref[...], preferred_element_type=F32)


def _kv_proj_kernel(x_ref, wdkv_ref, wuk_ref, wuv_ref, g_ref, b_ref,
                    ckv_ref, ka_ref, va_ref):
    ckv = jnp.dot(x_ref[...], wdkv_ref[...], preferred_element_type=F32)
    ckv_ref[...] = ckv
    mask = jax.lax.broadcasted_iota(jnp.int32, ckv.shape, 1) < KVPD
    cm = jnp.where(mask, ckv, 0.0)
    m = jnp.sum(cm, axis=-1, keepdims=True) * (1.0 / KVPD)
    d = jnp.where(mask, ckv - m, 0.0)
    v = jnp.sum(d * d, axis=-1, keepdims=True) * (1.0 / KVPD)
    kvn = (ckv - m) * jax.lax.rsqrt(v + EPS) * g_ref[...] + b_ref[...]
    ka_ref[...] = jnp.dot(kvn, wuk_ref[...], preferred_element_type=F32)
    va_ref[...] = jnp.dot(kvn, wuv_ref[...], preferred_element_type=F32)


def _attn_kernel(q_ref, ka_ref, va_ref, kr_ref, cos_ref, sin_ref,
                 o_ref, krr_ref):
    h = pl.program_id(0)
    i = pl.program_id(1)

    @pl.when(jnp.logical_and(h == 0, i == 0))
    def _():
        kr = kr_ref[...]
        krr_ref[...] = kr * cos_ref[...] + _rot_half(kr) * sin_ref[...]

    q = q_ref[...]
    qn = q[:, :NOPE]
    qr = q[:, NOPE:]
    cos_q = cos_ref[pl.ds(i * BQ, BQ), :]
    sin_q = sin_ref[pl.ds(i * BQ, BQ), :]
    qr = qr * cos_q + _rot_half(qr) * sin_q
    qh = jnp.concatenate([qn, qr], axis=-1)
    kh = jnp.concatenate([ka_ref[0], krr_ref[...]], axis=-1)
    logits = jax.lax.dot_general(
        qh, kh, (((1,), (1,)), ((), ())), preferred_element_type=F32) * SCALE
    mx = jnp.max(logits, axis=-1, keepdims=True)
    e = jnp.exp(logits - mx)
    s = jnp.sum(e, axis=-1, keepdims=True)
    acc = jnp.dot(e, va_ref[...], preferred_element_type=F32)
    o_ref[...] = acc / s


def _out_proj_kernel(a_ref, wo_ref, o_ref):
    o_ref[...] = jax.lax.dot_general(
        a_ref[...], wo_ref[...], (((1,), (1,)), ((), ())),
        preferred_element_type=F32)


def kernel(x, W_dq, W_uq, q_ln_g, q_ln_b, W_dkv, W_ukv, kv_ln_g, kv_ln_b, W_o):
    x2 = x.reshape(S, D)
    nI = S // BQ

    freqs = 1.0 / (10000.0 ** (jnp.arange(0, DH, 2, dtype=F32) / DH))
    emb = jnp.arange(S, dtype=F32)[:, None] * freqs[None, : RP // 2]
    cos_t = jnp.concatenate([jnp.cos(emb), jnp.cos(emb)], axis=-1)
    sin_t = jnp.concatenate([jnp.sin(emb), jnp.sin(emb)], axis=-1)

    w3 = W_ukv.reshape(KVPD, H, DH + NOPE)
    wuk = jnp.pad(w3[:, :, :NOPE].reshape(KVPD, H * NOPE), ((0, RP), (0, 0)))
    wuv = jnp.pad(w3[:, :, NOPE:].reshape(KVPD, H * DH), ((0, RP), (0, 0)))
    kv_g = jnp.pad(kv_ln_g, (0, RP))[None, :]
    kv_b = jnp.pad(kv_ln_b, (0, RP))[None, :]

    Q = pl.pallas_call(
        _q_proj_kernel,
        grid=(nI,),
        in_specs=[
            pl.BlockSpec((BQ, D), lambda i: (i, 0)),
            pl.BlockSpec((D, QPD), lambda i: (0, 0)),
            pl.BlockSpec((QPD, D), lambda i: (0, 0)),
            pl.BlockSpec((1, QPD), lambda i: (0, 0)),
            pl.BlockSpec((1, QPD), lambda i: (0, 0)),
        ],
        out_specs=pl.BlockSpec((BQ, D), lambda i: (i, 0)),
        out_shape=jax.ShapeDtypeStruct((S, D), F32),
    )(x2, W_dq, W_uq, q_ln_g[None, :], q_ln_b[None, :])

    ckv, KA, VA = pl.pallas_call(
        _kv_proj_kernel,
        grid=(nI,),
        in_specs=[
            pl.BlockSpec((BQ, D), lambda i: (i, 0)),
            pl.BlockSpec((D, CKV_W), lambda i: (0, 0)),
            pl.BlockSpec((CKV_W, H * NOPE), lambda i: (0, 0)),
            pl.BlockSpec((CKV_W, H * DH), lambda i: (0, 0)),
            pl.BlockSpec((1, CKV_W), lambda i: (0, 0)),
            pl.BlockSpec((1, CKV_W), lambda i: (0, 0)),
        ],
        out_specs=[
            pl.BlockSpec((BQ, CKV_W), lambda i: (i, 0)),
            pl.BlockSpec((BQ, H * NOPE), lambda i: (i, 0)),
            pl.BlockSpec((BQ, H * DH), lambda i: (i, 0)),
        ],
        out_shape=[
            jax.ShapeDtypeStruct((S, CKV_W), F32),
            jax.ShapeDtypeStruct((S, H * NOPE), F32),
            jax.ShapeDtypeStruct((S, H * DH), F32),
        ],
    )(x2, W_dkv, wuk, wuv, kv_g, kv_b)

    Kr = ckv[:, KVPD:]
    KA3 = KA.reshape(S, H, NOPE).transpose(1, 0, 2)

    attn = pl.pallas_call(
        _attn_kernel,
        grid=(H, nI),
        in_specs=[
            pl.BlockSpec((BQ, DH), lambda h, i: (i, h)),
            pl.BlockSpec((1, S, NOPE), lambda h, i: (h, 0, 0)),
            pl.BlockSpec((S, DH), lambda h, i: (0, h)),
            pl.BlockSpec((S, RP), lambda h, i: (0, 0)),
            pl.BlockSpec((S, RP), lambda h, i: (0, 0)),
            pl.BlockSpec((S, RP), lambda h, i: (0, 0)),
        ],
        out_specs=pl.BlockSpec((BQ, DH), lambda h, i: (i, h)),
        out_shape=jax.ShapeDtypeStruct((S, H * DH), F32),
        scratch_shapes=[pltpu.VMEM((S, RP), F32)],
    )(Q, KA3, VA, Kr, cos_t, sin_t)

    out = pl.pallas_call(
        _out_proj_kernel,
        grid=(nI,),
        in_specs=[
            pl.BlockSpec((BQ, D), lambda i: (i, 0)),
            pl.BlockSpec((D, D), lambda i: (0, 0)),
        ],
        out_specs=pl.BlockSpec((BQ, D), lambda i: (i, 0)),
        out_shape=jax.ShapeDtypeStruct((S, D), F32),
    )(attn, W_o)

    return (out.reshape(1, S, D), ckv.reshape(1, S, CKV_W))


# speedup vs baseline: 1.1393x; 1.1049x over previous
import jax
import jax.numpy as jnp
from jax.experimental import pallas as pl
from jax.experimental.pallas import tpu as pltpu

D = 2048
S = 2048
H = 16
DH = 128
NOPE = 64
RP = 64
QPD = 1024
KVPD = 1365
CKV_W = KVPD + RP
KV_OUT = D + H * NOPE
BQ = 256
EPS = 1e-5
SCALE = 1.0 / (DH ** 0.5)
F32 = jnp.float32
BF16 = jnp.bfloat16


def _rot_half(x):
    half = x.shape[-1] // 2
    return jnp.concatenate([-x[:, half:], x[:, :half]], axis=-1)


def _q_proj_kernel(x_ref, wdq_ref, wuq_ref, g_ref, b_ref, q_ref):
    cq = jnp.dot(x_ref[...].astype(BF16), wdq_ref[...].astype(BF16),
                 preferred_element_type=F32)
    m = jnp.mean(cq, axis=-1, keepdims=True)
    d = cq - m
    v = jnp.mean(d * d, axis=-1, keepdims=True)
    cqn = d * jax.lax.rsqrt(v + EPS) * g_ref[...] + b_ref[...]
    q_ref[...] = jnp.dot(cqn.astype(BF16), wuq_ref[...].astype(BF16),
                         preferred_element_type=F32)


def _kv_proj_kernel(x_ref, wdkv_ref, wuk_ref, wuv_ref, g_ref, b_ref,
                    ckv_ref, ka_ref, va_ref):
    ckv = jnp.dot(x_ref[...].astype(BF16), wdkv_ref[...].astype(BF16),
                  preferred_element_type=F32)
    ckv_ref[...] = ckv
    mask = jax.lax.broadcasted_iota(jnp.int32, ckv.shape, 1) < KVPD
    cm = jnp.where(mask, ckv, 0.0)
    m = jnp.sum(cm, axis=-1, keepdims=True) * (1.0 / KVPD)
    d = jnp.where(mask, ckv - m, 0.0)
    v = jnp.sum(d * d, axis=-1, keepdims=True) * (1.0 / KVPD)
    kvn = (ckv - m) * jax.lax.rsqrt(v + EPS) * g_ref[...] + b_ref[...]
    kvn = kvn.astype(BF16)
    ka_ref[...] = jnp.dot(kvn, wuk_ref[...].astype(BF16),
                          preferred_element_type=F32)
    va_ref[...] = jnp.dot(kvn, wuv_ref[...].astype(BF16),
                          preferred_element_type=F32)


def _attn_kernel(q_ref, ka_ref, va_ref, kr_ref, cos_ref, sin_ref,
                 o_ref, krr_ref):
    h = pl.program_id(0)
    i = pl.program_id(1)

    @pl.when(jnp.logical_and(h == 0, i == 0))
    def _():
        kr = kr_ref[...]
        krr_ref[...] = kr * cos_ref[...] + _rot_half(kr) * sin_ref[...]

    q = q_ref[...]
    qn = q[:, :NOPE]
    qr = q[:, NOPE:]
    cos_q = cos_ref[pl.ds(i * BQ, BQ), :]
    sin_q = sin_ref[pl.ds(i * BQ, BQ), :]
    qr = qr * cos_q + _rot_half(qr) * sin_q
    qh = jnp.concatenate([qn, qr], axis=-1).astype(BF16)
    kh = jnp.concatenate([ka_ref[0], krr_ref[...]], axis=-1).astype(BF16)
    logits = jax.lax.dot_general(
        qh, kh, (((1,), (1,)), ((), ())), preferred_element_type=F32) * SCALE
    mx = jnp.max(logits, axis=-1, keepdims=True)
    e = jnp.exp(logits - mx)
    s = jnp.sum(e, axis=-1, keepdims=True)
    acc = jnp.dot(e.astype(BF16), va_ref[...].astype(BF16),
                  preferred_element_type=F32)
    o_ref[...] = acc / s


def _out_proj_kernel(a_ref, wo_ref, o_ref):
    o_ref[...] = jax.lax.dot_general(
        a_ref[...].astype(BF16), wo_ref[...].astype(BF16),
        (((1,), (1,)), ((), ())), preferred_element_type=F32)


def kernel(x, W_dq, W_uq, q_ln_g, q_ln_b, W_dkv, W_ukv, kv_ln_g, kv_ln_b, W_o):
    x2 = x.reshape(S, D)
    nI = S // BQ

    freqs = 1.0 / (10000.0 ** (jnp.arange(0, DH, 2, dtype=F32) / DH))
    emb = jnp.arange(S, dtype=F32)[:, None] * freqs[None, : RP // 2]
    cos_t = jnp.concatenate([jnp.cos(emb), jnp.cos(emb)], axis=-1)
    sin_t = jnp.concatenate([jnp.sin(emb), jnp.sin(emb)], axis=-1)

    w3 = W_ukv.reshape(KVPD, H, DH + NOPE)
    wuk = jnp.pad(w3[:, :, :NOPE].reshape(KVPD, H * NOPE), ((0, RP), (0, 0)))
    wuv = jnp.pad(w3[:, :, NOPE:].reshape(KVPD, H * DH), ((0, RP), (0, 0)))
    kv_g = jnp.pad(kv_ln_g, (0, RP))[None, :]
    kv_b = jnp.pad(kv_ln_b, (0, RP))[None, :]

    Q = pl.pallas_call(
        _q_proj_kernel,
        grid=(nI,),
        in_specs=[
            pl.BlockSpec((BQ, D), lambda i: (i, 0)),
            pl.BlockSpec((D, QPD), lambda i: (0, 0)),
            pl.BlockSpec((QPD, D), lambda i: (0, 0)),
            pl.BlockSpec((1, QPD), lambda i: (0, 0)),
            pl.BlockSpec((1, QPD), lambda i: (0, 0)),
        ],
        out_specs=pl.BlockSpec((BQ, D), lambda i: (i, 0)),
        out_shape=jax.ShapeDtypeStruct((S, D), F32),
    )(x2, W_dq, W_uq, q_ln_g[None, :], q_ln_b[None, :])

    ckv, KA, VA = pl.pallas_call(
        _kv_proj_kernel,
        grid=(nI,),
        in_specs=[
            pl.BlockSpec((BQ, D), lambda i: (i, 0)),
            pl.BlockSpec((D, CKV_W), lambda i: (0, 0)),
            pl.BlockSpec((CKV_W, H * NOPE), lambda i: (0, 0)),
            pl.BlockSpec((CKV_W, H * DH), lambda i: (0, 0)),
            pl.BlockSpec((1, CKV_W), lambda i: (0, 0)),
            pl.BlockSpec((1, CKV_W), lambda i: (0, 0)),
        ],
        out_specs=[
            pl.BlockSpec((BQ, CKV_W), lambda i: (i, 0)),
            pl.BlockSpec((BQ, H * NOPE), lambda i: (i, 0)),
            pl.BlockSpec((BQ, H * DH), lambda i: (i, 0)),
        ],
        out_shape=[
            jax.ShapeDtypeStruct((S, CKV_W), F32),
            jax.ShapeDtypeStruct((S, H * NOPE), F32),
            jax.ShapeDtypeStruct((S, H * DH), F32),
        ],
    )(x2, W_dkv, wuk, wuv, kv_g, kv_b)

    Kr = ckv[:, KVPD:]
    KA3 = KA.reshape(S, H, NOPE).transpose(1, 0, 2)

    attn = pl.pallas_call(
        _attn_kernel,
        grid=(H, nI),
        in_specs=[
            pl.BlockSpec((BQ, DH), lambda h, i: (i, h)),
            pl.BlockSpec((1, S, NOPE), lambda h, i: (h, 0, 0)),
            pl.BlockSpec((S, DH), lambda h, i: (0, h)),
            pl.BlockSpec((S, RP), lambda h, i: (0, 0)),
            pl.BlockSpec((S, RP), lambda h, i: (0, 0)),
            pl.BlockSpec((S, RP), lambda h, i: (0, 0)),
        ],
        out_specs=pl.BlockSpec((BQ, DH), lambda h, i: (i, h)),
        out_shape=jax.ShapeDtypeStruct((S, H * DH), F32),
        scratch_shapes=[pltpu.VMEM((S, RP), F32)],
    )(Q, KA3, VA, Kr, cos_t, sin_t)

    out = pl.pallas_call(
        _out_proj_kernel,
        grid=(nI,),
        in_specs=[
            pl.BlockSpec((BQ, D), lambda i: (i, 0)),
            pl.BlockSpec((D, D), lambda i: (0, 0)),
        ],
        out_specs=pl.BlockSpec((BQ, D), lambda i: (i, 0)),
        out_shape=jax.ShapeDtypeStruct((S, D), F32),
    )(attn, W_o)

    return (out.reshape(1, S, D), ckv.reshape(1, S, CKV_W))


# speedup vs baseline: 1.2814x; 1.1247x over previous
import jax
import jax.numpy as jnp
from jax.experimental import pallas as pl
from jax.experimental.pallas import tpu as pltpu

D = 2048
S = 2048
H = 16
DH = 128
NOPE = 64
RP = 64
QPD = 1024
KVPD = 1365
CKV_W = KVPD + RP
BQ = 256
EPS = 1e-5
SCALE = 1.0 / (DH ** 0.5)
LOG2E = 1.4426950408889634
F32 = jnp.float32
BF16 = jnp.bfloat16


def _rot_rope(x3):
    z = jnp.zeros_like(x3[..., :NOPE])
    return jnp.concatenate(
        [z, -x3[..., NOPE + RP // 2:], x3[..., NOPE:NOPE + RP // 2]], axis=-1)


def _q_proj_kernel(x_ref, wdq_ref, wuq_ref, g_ref, b_ref, cos_ref, sin_ref,
                   q_ref):
    cq = jnp.dot(x_ref[...], wdq_ref[...], preferred_element_type=F32)
    m = jnp.mean(cq, axis=-1, keepdims=True)
    d = cq - m
    v = jnp.mean(d * d, axis=-1, keepdims=True)
    cqn = d * jax.lax.rsqrt(v + EPS) * g_ref[...] + b_ref[...]
    q = jnp.dot(cqn.astype(BF16), wuq_ref[...], preferred_element_type=F32)
    q3 = q.reshape(BQ, H, DH)
    qh = q3 * cos_ref[...][:, None, :] + _rot_rope(q3) * sin_ref[...][:, None, :]
    q_ref[...] = qh.reshape(BQ, D).astype(BF16)


def _kv_proj_kernel(x_ref, wdkv_ref, wukb_ref, wuv_ref, g_ref, b_ref,
                    cos_ref, sin_ref, ckv_ref, kb_ref, va_ref):
    o = jnp.dot(x_ref[...], wdkv_ref[...], preferred_element_type=F32)
    kr = o[:, :DH]
    ckv = o[:, DH:]
    ckv_ref[...] = ckv
    mask = jax.lax.broadcasted_iota(jnp.int32, ckv.shape, 1) < KVPD
    cm = jnp.where(mask, ckv, 0.0)
    m = jnp.sum(cm, axis=-1, keepdims=True) * (1.0 / KVPD)
    d = jnp.where(mask, ckv - m, 0.0)
    v = jnp.sum(d * d, axis=-1, keepdims=True) * (1.0 / KVPD)
    kvn = ((ckv - m) * jax.lax.rsqrt(v + EPS) * g_ref[...]
           + b_ref[...]).astype(BF16)
    krr = kr * cos_ref[...] + _rot_rope(kr) * sin_ref[...]
    kb = jnp.dot(kvn, wukb_ref[...], preferred_element_type=F32)
    kb = kb + jnp.concatenate([krr] * H, axis=-1)
    kb_ref[...] = kb.astype(BF16)
    va_ref[...] = jnp.dot(kvn, wuv_ref[...],
                          preferred_element_type=F32).astype(BF16)


def _attn_kernel(q_ref, kb_ref, va_ref, o_ref):
    logits = jax.lax.dot_general(
        q_ref[...], kb_ref[...], (((1,), (1,)), ((), ())),
        preferred_element_type=F32)
    e = jnp.exp2(logits.astype(BF16))
    s = jnp.sum(e.astype(F32), axis=-1, keepdims=True)
    acc = jnp.dot(e, va_ref[...], preferred_element_type=F32)
    o_ref[...] = (acc / s).astype(BF16)


def _out_proj_kernel(a_ref, wo_ref, o_ref):
    o_ref[...] = jax.lax.dot_general(
        a_ref[...], wo_ref[...], (((1,), (1,)), ((), ())),
        preferred_element_type=F32)


def kernel(x, W_dq, W_uq, q_ln_g, q_ln_b, W_dkv, W_ukv, kv_ln_g, kv_ln_b, W_o):
    x2 = x.reshape(S, D).astype(BF16)
    nI = S // BQ

    freqs = 1.0 / (10000.0 ** (jnp.arange(0, DH, 2, dtype=F32) / DH))
    emb = jnp.arange(S, dtype=F32)[:, None] * freqs[None, : RP // 2]
    cos64 = jnp.tile(jnp.cos(emb), (1, 2))
    sin64 = jnp.tile(jnp.sin(emb), (1, 2))
    ones64 = jnp.ones((S, NOPE), F32)
    zeros64 = jnp.zeros((S, NOPE), F32)
    qs = SCALE * LOG2E
    cosq = qs * jnp.concatenate([ones64, cos64], axis=-1)
    sinq = qs * jnp.concatenate([zeros64, sin64], axis=-1)
    cosk = jnp.concatenate([ones64, cos64], axis=-1)
    sink = jnp.concatenate([zeros64, sin64], axis=-1)

    wdq = W_dq.astype(BF16)
    wuq = W_uq.astype(BF16)
    wkr = jnp.pad(W_dkv[:, KVPD:], ((0, 0), (NOPE, 0)))
    wdkv_ext = jnp.concatenate([wkr, W_dkv], axis=-1).astype(BF16)
    w3 = W_ukv.reshape(KVPD, H, DH + NOPE)
    wukb = jnp.pad(w3[:, :, :NOPE],
                   ((0, RP), (0, 0), (0, RP))).reshape(CKV_W, H * DH)
    wukb = wukb.astype(BF16)
    wuv = jnp.pad(w3[:, :, NOPE:].reshape(KVPD, H * DH),
                  ((0, RP), (0, 0))).astype(BF16)
    wo = W_o.astype(BF16)
    kv_g = jnp.pad(kv_ln_g, (0, RP))[None, :]
    kv_b = jnp.pad(kv_ln_b, (0, RP))[None, :]

    Q = pl.pallas_call(
        _q_proj_kernel,
        grid=(nI,),
        in_specs=[
            pl.BlockSpec((BQ, D), lambda i: (i, 0)),
            pl.BlockSpec((D, QPD), lambda i: (0, 0)),
            pl.BlockSpec((QPD, D), lambda i: (0, 0)),
            pl.BlockSpec((1, QPD), lambda i: (0, 0)),
            pl.BlockSpec((1, QPD), lambda i: (0, 0)),
            pl.BlockSpec((BQ, DH), lambda i: (i, 0)),
            pl.BlockSpec((BQ, DH), lambda i: (i, 0)),
        ],
        out_specs=pl.BlockSpec((BQ, D), lambda i: (i, 0)),
        out_shape=jax.ShapeDtypeStruct((S, D), BF16),
    )(x2, wdq, wuq, q_ln_g[None, :], q_ln_b[None, :], cosq, sinq)

    ckv, KB, VA = pl.pallas_call(
        _kv_proj_kernel,
        grid=(nI,),
        in_specs=[
            pl.BlockSpec((BQ, D), lambda i: (i, 0)),
            pl.BlockSpec((D, DH + CKV_W), lambda i: (0, 0)),
            pl.BlockSpec((CKV_W, H * DH), lambda i: (0, 0)),
            pl.BlockSpec((CKV_W, H * DH), lambda i: (0, 0)),
            pl.BlockSpec((1, CKV_W), lambda i: (0, 0)),
            pl.BlockSpec((1, CKV_W), lambda i: (0, 0)),
            pl.BlockSpec((BQ, DH), lambda i: (i, 0)),
            pl.BlockSpec((BQ, DH), lambda i: (i, 0)),
        ],
        out_specs=[
            pl.BlockSpec((BQ, CKV_W), lambda i: (i, 0)),
            pl.BlockSpec((BQ, H * DH), lambda i: (i, 0)),
            pl.BlockSpec((BQ, H * DH), lambda i: (i, 0)),
        ],
        out_shape=[
            jax.ShapeDtypeStruct((S, CKV_W), F32),
            jax.ShapeDtypeStruct((S, H * DH), BF16),
            jax.ShapeDtypeStruct((S, H * DH), BF16),
        ],
    )(x2, wdkv_ext, wukb, wuv, kv_g, kv_b, cosk, sink)

    attn = pl.pallas_call(
        _attn_kernel,
        grid=(H, nI),
        in_specs=[
            pl.BlockSpec((BQ, DH), lambda h, i: (i, h)),
            pl.BlockSpec((S, DH), lambda h, i: (0, h)),
            pl.BlockSpec((S, DH), lambda h, i: (0, h)),
        ],
        out_specs=pl.BlockSpec((BQ, DH), lambda h, i: (i, h)),
        out_shape=jax.ShapeDtypeStruct((S, H * DH), BF16),
    )(Q, KB, VA)

    out = pl.pallas_call(
        _out_proj_kernel,
        grid=(nI,),
        in_specs=[
            pl.BlockSpec((BQ, D), lambda i: (i, 0)),
            pl.BlockSpec((D, D), lambda i: (0, 0)),
        ],
        out_specs=pl.BlockSpec((BQ, D), lambda i: (i, 0)),
        out_shape=jax.ShapeDtypeStruct((S, D), F32),
    )(attn, wo)

    return (out.reshape(1, S, D), ckv.reshape(1, S, CKV_W))
